# baseline (speedup 1.0000x reference)
import functools

import jax
import jax.numpy as jnp
from jax import lax
from jax.experimental import pallas as pl
from jax.experimental.pallas import tpu as pltpu
from jax.experimental.pallas import tpu_sc as plsc

N = 10000
E = 320000
D = 128
T = 4
G = 100
STEPS = 8
HID = 256

NC = 2
NS = 16
NW = NC * NS
EPW = E // NW
CH = 128
CHB = 2
EPC = CHB * CH
NH = 2
CPH = 20
CPW = NH * CPH
PAD = CPW * EPC - EPW
NACC = 10240
ZCH = NACC // NS // CH
ORT = NACC // NS

BN = 1000
NBLK = N // BN


def _table_tc(h, W_et, b_et):
    def body(h_ref, w_ref, b_ref, o_ref):
        o_ref[...] = lax.dot_general(
            h_ref[...], w_ref[0],
            (((1,), (1,)), ((), ())),
            preferred_element_type=jnp.float32) + b_ref[0]

    return pl.pallas_call(
        body,
        grid=(NBLK, T),
        in_specs=[
            pl.BlockSpec((BN, D), lambda i, t: (i, 0)),
            pl.BlockSpec((1, D, D), lambda i, t: (t, 0, 0)),
            pl.BlockSpec((1, 1, D), lambda i, t: (t, 0, 0)),
        ],
        out_specs=pl.BlockSpec((BN, D), lambda i, t: (t * NBLK + i, 0)),
        out_shape=jax.ShapeDtypeStruct((T * N, D), jnp.float32),
    )(h, W_et, b_et)


def _edge_sc(table, flat_w, dst_w, zeros_chunk):
    mesh = plsc.VectorSubcoreMesh(core_axis_name="c", subcore_axis_name="s")

    @functools.partial(
        pl.kernel,
        out_type=jax.ShapeDtypeStruct((NC, NACC, D), jnp.float32),
        mesh=mesh,
        scratch_types=[
            pltpu.VMEM((CPH * EPC,), jnp.int32),
            pltpu.VMEM((CPH * EPC,), jnp.int32),
            pltpu.VMEM((EPC, D), jnp.float32),
            pltpu.VMEM_SHARED((NACC, D), jnp.float32),
            pltpu.SemaphoreType.DMA,
        ],
    )
    def run(table_h, flat_h, dst_h, zero_h, out_h, fhalf, dhalf, buf_v,
            acc_s, gsem):
        c = lax.axis_index("c")
        s = lax.axis_index("s")
        wid = s * NC + c

        pltpu.sync_copy(zero_h, buf_v.at[pl.ds(0, CH)])
        for z in range(ZCH):
            pltpu.sync_copy(buf_v.at[pl.ds(0, CH)],
                            acc_s.at[pl.ds(s * ORT + z * CH, CH)])
        plsc.subcore_barrier()

        for half in range(NH):
            pltpu.sync_copy(flat_h.at[wid, half], fhalf)
            pltpu.sync_copy(dst_h.at[wid, half], dhalf)

            def chunk(m, carry):
                off = pl.multiple_of(m * EPC, EPC)
                pltpu.async_copy(table_h.at[fhalf.at[pl.ds(off, EPC)]],
                                 buf_v, gsem).wait()
                pltpu.sync_copy(buf_v, acc_s.at[dhalf.at[pl.ds(off, EPC)]],
                                add=True)
                return carry

            lax.fori_loop(0, CPH, chunk, 0)
        plsc.subcore_barrier()

        pltpu.sync_copy(acc_s.at[pl.ds(s * ORT, ORT)],
                        out_h.at[c, pl.ds(s * ORT, ORT)])

    return run(table, flat_w, dst_w, zeros_chunk)


def _gru_tc(aparts, h, W_ih, W_hh, b_ih2, b_hh2):
    def body(a_ref, h_ref, wi_ref, wh_ref, bi_ref, bh_ref, o_ref):
        a = a_ref[0] + a_ref[1]
        hb = h_ref[...]
        gi = lax.dot_general(a, wi_ref[...], (((1,), (1,)), ((), ())),
                             preferred_element_type=jnp.float32) + bi_ref[...]
        gh = lax.dot_general(hb, wh_ref[...], (((1,), (1,)), ((), ())),
                             preferred_element_type=jnp.float32) + bh_ref[...]
        r = jax.nn.sigmoid(gi[:, :D] + gh[:, :D])
        z = jax.nn.sigmoid(gi[:, D:2 * D] + gh[:, D:2 * D])
        n = jnp.tanh(gi[:, 2 * D:] + r * gh[:, 2 * D:])
        o_ref[...] = (1.0 - z) * n + z * hb

    return pl.pallas_call(
        body,
        grid=(NBLK,),
        in_specs=[
            pl.BlockSpec((NC, BN, D), lambda i: (0, i, 0)),
            pl.BlockSpec((BN, D), lambda i: (i, 0)),
            pl.BlockSpec((3 * D, D), lambda i: (0, 0)),
            pl.BlockSpec((3 * D, D), lambda i: (0, 0)),
            pl.BlockSpec((1, 3 * D), lambda i: (0, 0)),
            pl.BlockSpec((1, 3 * D), lambda i: (0, 0)),
        ],
        out_specs=pl.BlockSpec((BN, D), lambda i: (i, 0)),
        out_shape=jax.ShapeDtypeStruct((N, D), jnp.float32),
    )(aparts, h, W_ih, W_hh, b_ih2, b_hh2)


def _pool_mlp_tc(h, gid2, W1, b1_2, W2p, b2r):
    def body(h_ref, g_ref, w1_ref, b1_ref, w2_ref, b2_ref, o_ref, sums, cnts):
        i = pl.program_id(0)
        ids = g_ref[...]
        M = (ids == lax.broadcasted_iota(jnp.int32, (BN, D), 1)
             ).astype(jnp.float32)
        hb = h_ref[...]
        ps = lax.dot_general(M, hb, (((0,), (0,)), ((), ())),
                             preferred_element_type=jnp.float32)
        pc = lax.dot_general(M, jnp.ones((BN, D), jnp.float32),
                             (((0,), (0,)), ((), ())),
                             preferred_element_type=jnp.float32)

        @pl.when(i == 0)
        def _():
            sums[...] = ps
            cnts[...] = pc

        @pl.when(i > 0)
        def _():
            sums[...] += ps
            cnts[...] += pc

        @pl.when(i == NBLK - 1)
        def _():
            mean = sums[...] / jnp.maximum(cnts[...], 1.0)
            hid = jax.nn.relu(
                lax.dot_general(mean, w1_ref[...], (((1,), (1,)), ((), ())),
                                preferred_element_type=jnp.float32)
                + b1_ref[...])
            logit = lax.dot_general(hid, w2_ref[...], (((1,), (1,)), ((), ())),
                                    preferred_element_type=jnp.float32)
            o_ref[...] = jax.nn.sigmoid(logit + b2_ref[...])

    return pl.pallas_call(
        body,
        grid=(NBLK,),
        in_specs=[
            pl.BlockSpec((BN, D), lambda i: (i, 0)),
            pl.BlockSpec((BN, 1), lambda i: (i, 0)),
            pl.BlockSpec((HID, D), lambda i: (0, 0)),
            pl.BlockSpec((1, HID), lambda i: (0, 0)),
            pl.BlockSpec((D, HID), lambda i: (0, 0)),
            pl.BlockSpec((1, D), lambda i: (0, 0)),
        ],
        out_specs=pl.BlockSpec((D, D), lambda i: (0, 0)),
        out_shape=jax.ShapeDtypeStruct((D, D), jnp.float32),
        scratch_shapes=[
            pltpu.VMEM((D, D), jnp.float32),
            pltpu.VMEM((D, D), jnp.float32),
        ],
    )(h, gid2, W1, b1_2, W2p, b2r)


def kernel(x, edge_index, edge_types, graph_ids, W_et, b_et, W_ih, W_hh,
           b_ih, b_hh, W1, b1, W2, b2):
    src = edge_index[0]
    dst = edge_index[1]
    flat = edge_types * N + src

    flat_w = jnp.pad(flat.reshape(NW, EPW), ((0, 0), (0, PAD)),
                     constant_values=0).reshape(NW, NH, CPH * EPC)
    dst_w = jnp.pad(dst.reshape(NW, EPW), ((0, 0), (0, PAD)),
                    constant_values=N).reshape(NW, NH, CPH * EPC)
    zeros_chunk = jnp.zeros((CH, D), jnp.float32)

    b_et3 = b_et.reshape(T, 1, D)
    b_ih2 = b_ih.reshape(1, 3 * D)
    b_hh2 = b_hh.reshape(1, 3 * D)
    b1_2 = b1.reshape(1, HID)
    gid2 = graph_ids.reshape(N, 1)
    W2p = jnp.zeros((D, HID), jnp.float32).at[0].set(W2[0])
    b2r = jnp.broadcast_to(b2.reshape(1, 1), (1, D))

    h = x
    for _ in range(STEPS):
        table = _table_tc(h, W_et, b_et3)
        aparts = _edge_sc(table, flat_w, dst_w, zeros_chunk)
        h = _gru_tc(aparts, h, W_ih, W_hh, b_ih2, b_hh2)

    out_full = _pool_mlp_tc(h, gid2, W1, b1_2, W2p, b2r)
    return out_full[:G, :1]


# speedup vs baseline: 1.4718x; 1.4718x over previous
import functools

import jax
import jax.numpy as jnp
from jax import lax
from jax.experimental import pallas as pl
from jax.experimental.pallas import tpu as pltpu
from jax.experimental.pallas import tpu_sc as plsc

N = 10000
E = 320000
D = 128
T = 4
G = 100
STEPS = 8
HID = 256

NC = 2
NS = 16
NW = NC * NS
EPW = E // NW
CH = 128
CPW = (EPW + CH - 1) // CH
PAD = CPW * CH - EPW
NACC = 10240
ZCH = NACC // NS // CH
ORT = NACC // NS

BN = 1000
NBLK = N // BN


def _table_tc(h, W_et, b_et):
    def body(h_ref, w_ref, b_ref, o_ref):
        o_ref[...] = lax.dot_general(
            h_ref[...], w_ref[0],
            (((1,), (1,)), ((), ())),
            preferred_element_type=jnp.float32) + b_ref[0]

    return pl.pallas_call(
        body,
        grid=(NBLK, T),
        in_specs=[
            pl.BlockSpec((BN, D), lambda i, t: (i, 0)),
            pl.BlockSpec((1, D, D), lambda i, t: (t, 0, 0)),
            pl.BlockSpec((1, 1, D), lambda i, t: (t, 0, 0)),
        ],
        out_specs=pl.BlockSpec((BN, D), lambda i, t: (t * NBLK + i, 0)),
        out_shape=jax.ShapeDtypeStruct((T * N, D), jnp.float32),
    )(h, W_et, b_et)


def _edge_sc(table, flat_w, dst_w, zeros_chunk):
    mesh = plsc.VectorSubcoreMesh(core_axis_name="c", subcore_axis_name="s")

    @functools.partial(
        pl.kernel,
        out_type=jax.ShapeDtypeStruct((NC, NACC, D), jnp.float32),
        mesh=mesh,
        scratch_types=[
            pltpu.VMEM((CPW, CH), jnp.int32),
            pltpu.VMEM((CPW, CH), jnp.int32),
            pltpu.VMEM((CH, D), jnp.float32),
            pltpu.VMEM_SHARED((NACC, D), jnp.float32),
            pltpu.SemaphoreType.DMA,
        ],
    )
    def run(table_h, flat_h, dst_h, zero_h, out_h, idx_v, dst_v, buf_v,
            acc_s, sem):
        c = lax.axis_index("c")
        s = lax.axis_index("s")
        wid = s * NC + c

        pltpu.sync_copy(zero_h, buf_v)
        for z in range(ZCH):
            pltpu.sync_copy(buf_v, acc_s.at[pl.ds(s * ORT + z * CH, CH)])
        plsc.subcore_barrier()

        pltpu.sync_copy(flat_h.at[wid], idx_v)
        pltpu.sync_copy(dst_h.at[wid], dst_v)

        def chunk(j, carry):
            pltpu.async_copy(table_h.at[idx_v.at[j]], buf_v, sem).wait()
            pltpu.sync_copy(buf_v, acc_s.at[dst_v.at[j]], add=True)
            return carry

        lax.fori_loop(0, CPW, chunk, 0)
        plsc.subcore_barrier()

        pltpu.sync_copy(acc_s.at[pl.ds(s * ORT, ORT)],
                        out_h.at[c, pl.ds(s * ORT, ORT)])

    return run(table, flat_w, dst_w, zeros_chunk)


def _gru_tc(aparts, h, W_ih, W_hh, b_ih2, b_hh2):
    def body(a_ref, h_ref, wi_ref, wh_ref, bi_ref, bh_ref, o_ref):
        a = a_ref[0] + a_ref[1]
        hb = h_ref[...]
        gi = lax.dot_general(a, wi_ref[...], (((1,), (1,)), ((), ())),
                             preferred_element_type=jnp.float32) + bi_ref[...]
        gh = lax.dot_general(hb, wh_ref[...], (((1,), (1,)), ((), ())),
                             preferred_element_type=jnp.float32) + bh_ref[...]
        r = jax.nn.sigmoid(gi[:, :D] + gh[:, :D])
        z = jax.nn.sigmoid(gi[:, D:2 * D] + gh[:, D:2 * D])
        n = jnp.tanh(gi[:, 2 * D:] + r * gh[:, 2 * D:])
        o_ref[...] = (1.0 - z) * n + z * hb

    return pl.pallas_call(
        body,
        grid=(NBLK,),
        in_specs=[
            pl.BlockSpec((NC, BN, D), lambda i: (0, i, 0)),
            pl.BlockSpec((BN, D), lambda i: (i, 0)),
            pl.BlockSpec((3 * D, D), lambda i: (0, 0)),
            pl.BlockSpec((3 * D, D), lambda i: (0, 0)),
            pl.BlockSpec((1, 3 * D), lambda i: (0, 0)),
            pl.BlockSpec((1, 3 * D), lambda i: (0, 0)),
        ],
        out_specs=pl.BlockSpec((BN, D), lambda i: (i, 0)),
        out_shape=jax.ShapeDtypeStruct((N, D), jnp.float32),
    )(aparts, h, W_ih, W_hh, b_ih2, b_hh2)


def _pool_mlp_tc(h, gid2, W1, b1_2, W2p, b2r):
    def body(h_ref, g_ref, w1_ref, b1_ref, w2_ref, b2_ref, o_ref, sums, cnts):
        i = pl.program_id(0)
        ids = g_ref[...]
        M = (ids == lax.broadcasted_iota(jnp.int32, (BN, D), 1)
             ).astype(jnp.float32)
        hb = h_ref[...]
        ps = lax.dot_general(M, hb, (((0,), (0,)), ((), ())),
                             preferred_element_type=jnp.float32)
        pc = lax.dot_general(M, jnp.ones((BN, D), jnp.float32),
                             (((0,), (0,)), ((), ())),
                             preferred_element_type=jnp.float32)

        @pl.when(i == 0)
        def _():
            sums[...] = ps
            cnts[...] = pc

        @pl.when(i > 0)
        def _():
            sums[...] += ps
            cnts[...] += pc

        @pl.when(i == NBLK - 1)
        def _():
            mean = sums[...] / jnp.maximum(cnts[...], 1.0)
            hid = jax.nn.relu(
                lax.dot_general(mean, w1_ref[...], (((1,), (1,)), ((), ())),
                                preferred_element_type=jnp.float32)
                + b1_ref[...])
            logit = lax.dot_general(hid, w2_ref[...], (((1,), (1,)), ((), ())),
                                    preferred_element_type=jnp.float32)
            o_ref[...] = jax.nn.sigmoid(logit + b2_ref[...])

    return pl.pallas_call(
        body,
        grid=(NBLK,),
        in_specs=[
            pl.BlockSpec((BN, D), lambda i: (i, 0)),
            pl.BlockSpec((BN, 1), lambda i: (i, 0)),
            pl.BlockSpec((HID, D), lambda i: (0, 0)),
            pl.BlockSpec((1, HID), lambda i: (0, 0)),
            pl.BlockSpec((D, HID), lambda i: (0, 0)),
            pl.BlockSpec((1, D), lambda i: (0, 0)),
        ],
        out_specs=pl.BlockSpec((D, D), lambda i: (0, 0)),
        out_shape=jax.ShapeDtypeStruct((D, D), jnp.float32),
        scratch_shapes=[
            pltpu.VMEM((D, D), jnp.float32),
            pltpu.VMEM((D, D), jnp.float32),
        ],
    )(h, gid2, W1, b1_2, W2p, b2r)


def _gru_table_tc(aparts, h, W_ih, W_hh, b_ih2, b_hh2, W_et, b_et3):
    def body(a_ref, h_ref, wi_ref, wh_ref, bi_ref, bh_ref, we_ref, be_ref,
             oh_ref, ot_ref):
        a = a_ref[0] + a_ref[1]
        hb = h_ref[...]
        gi = lax.dot_general(a, wi_ref[...], (((1,), (1,)), ((), ())),
                             preferred_element_type=jnp.float32) + bi_ref[...]
        gh = lax.dot_general(hb, wh_ref[...], (((1,), (1,)), ((), ())),
                             preferred_element_type=jnp.float32) + bh_ref[...]
        r = jax.nn.sigmoid(gi[:, :D] + gh[:, :D])
        z = jax.nn.sigmoid(gi[:, D:2 * D] + gh[:, D:2 * D])
        n = jnp.tanh(gi[:, 2 * D:] + r * gh[:, 2 * D:])
        hp = (1.0 - z) * n + z * hb
        oh_ref[...] = hp
        for t in range(T):
            ot_ref[t] = lax.dot_general(
                hp, we_ref[t], (((1,), (1,)), ((), ())),
                preferred_element_type=jnp.float32) + be_ref[t]

    return pl.pallas_call(
        body,
        grid=(NBLK,),
        in_specs=[
            pl.BlockSpec((NC, BN, D), lambda i: (0, i, 0)),
            pl.BlockSpec((BN, D), lambda i: (i, 0)),
            pl.BlockSpec((3 * D, D), lambda i: (0, 0)),
            pl.BlockSpec((3 * D, D), lambda i: (0, 0)),
            pl.BlockSpec((1, 3 * D), lambda i: (0, 0)),
            pl.BlockSpec((1, 3 * D), lambda i: (0, 0)),
            pl.BlockSpec((T, D, D), lambda i: (0, 0, 0)),
            pl.BlockSpec((T, 1, D), lambda i: (0, 0, 0)),
        ],
        out_specs=[
            pl.BlockSpec((BN, D), lambda i: (i, 0)),
            pl.BlockSpec((T, BN, D), lambda i: (0, i, 0)),
        ],
        out_shape=[
            jax.ShapeDtypeStruct((N, D), jnp.float32),
            jax.ShapeDtypeStruct((T, N, D), jnp.float32),
        ],
    )(aparts, h, W_ih, W_hh, b_ih2, b_hh2, W_et, b_et3)


def _gru_pool_tc(aparts, h, W_ih, W_hh, b_ih2, b_hh2, gid2, W1, b1_2, W2p,
                 b2r):
    def body(a_ref, h_ref, wi_ref, wh_ref, bi_ref, bh_ref, g_ref, w1_ref,
             b1_ref, w2_ref, b2_ref, o_ref, sums, cnts):
        i = pl.program_id(0)
        a = a_ref[0] + a_ref[1]
        hb = h_ref[...]
        gi = lax.dot_general(a, wi_ref[...], (((1,), (1,)), ((), ())),
                             preferred_element_type=jnp.float32) + bi_ref[...]
        gh = lax.dot_general(hb, wh_ref[...], (((1,), (1,)), ((), ())),
                             preferred_element_type=jnp.float32) + bh_ref[...]
        r = jax.nn.sigmoid(gi[:, :D] + gh[:, :D])
        z = jax.nn.sigmoid(gi[:, D:2 * D] + gh[:, D:2 * D])
        n = jnp.tanh(gi[:, 2 * D:] + r * gh[:, 2 * D:])
        hp = (1.0 - z) * n + z * hb

        M = (g_ref[...] == lax.broadcasted_iota(jnp.int32, (BN, D), 1)
             ).astype(jnp.float32)
        ps = lax.dot_general(M, hp, (((0,), (0,)), ((), ())),
                             preferred_element_type=jnp.float32)
        pc = lax.dot_general(M, jnp.ones((BN, D), jnp.float32),
                             (((0,), (0,)), ((), ())),
                             preferred_element_type=jnp.float32)

        @pl.when(i == 0)
        def _():
            sums[...] = ps
            cnts[...] = pc

        @pl.when(i > 0)
        def _():
            sums[...] += ps
            cnts[...] += pc

        @pl.when(i == NBLK - 1)
        def _():
            mean = sums[...] / jnp.maximum(cnts[...], 1.0)
            hid = jax.nn.relu(
                lax.dot_general(mean, w1_ref[...], (((1,), (1,)), ((), ())),
                                preferred_element_type=jnp.float32)
                + b1_ref[...])
            logit = lax.dot_general(hid, w2_ref[...], (((1,), (1,)), ((), ())),
                                    preferred_element_type=jnp.float32)
            o_ref[...] = jax.nn.sigmoid(logit + b2_ref[...])

    return pl.pallas_call(
        body,
        grid=(NBLK,),
        in_specs=[
            pl.BlockSpec((NC, BN, D), lambda i: (0, i, 0)),
            pl.BlockSpec((BN, D), lambda i: (i, 0)),
            pl.BlockSpec((3 * D, D), lambda i: (0, 0)),
            pl.BlockSpec((3 * D, D), lambda i: (0, 0)),
            pl.BlockSpec((1, 3 * D), lambda i: (0, 0)),
            pl.BlockSpec((1, 3 * D), lambda i: (0, 0)),
            pl.BlockSpec((BN, 1), lambda i: (i, 0)),
            pl.BlockSpec((HID, D), lambda i: (0, 0)),
            pl.BlockSpec((1, HID), lambda i: (0, 0)),
            pl.BlockSpec((D, HID), lambda i: (0, 0)),
            pl.BlockSpec((1, D), lambda i: (0, 0)),
        ],
        out_specs=pl.BlockSpec((D, D), lambda i: (0, 0)),
        out_shape=jax.ShapeDtypeStruct((D, D), jnp.float32),
        scratch_shapes=[
            pltpu.VMEM((D, D), jnp.float32),
            pltpu.VMEM((D, D), jnp.float32),
        ],
    )(aparts, h, W_ih, W_hh, b_ih2, b_hh2, gid2, W1, b1_2, W2p, b2r)


def kernel(x, edge_index, edge_types, graph_ids, W_et, b_et, W_ih, W_hh,
           b_ih, b_hh, W1, b1, W2, b2):
    src = edge_index[0]
    dst = edge_index[1]
    flat = edge_types * N + src

    flat_w = jnp.pad(flat.reshape(NW, EPW), ((0, 0), (0, PAD)),
                     constant_values=0).reshape(NW, CPW, CH)
    dst_w = jnp.pad(dst.reshape(NW, EPW), ((0, 0), (0, PAD)),
                    constant_values=N).reshape(NW, CPW, CH)
    zeros_chunk = jnp.zeros((CH, D), jnp.float32)

    b_et3 = b_et.reshape(T, 1, D)
    b_ih2 = b_ih.reshape(1, 3 * D)
    b_hh2 = b_hh.reshape(1, 3 * D)
    b1_2 = b1.reshape(1, HID)
    gid2 = graph_ids.reshape(N, 1)
    W2p = jnp.zeros((D, HID), jnp.float32).at[0].set(W2[0])
    b2r = jnp.broadcast_to(b2.reshape(1, 1), (1, D))

    h = x
    table = _table_tc(h, W_et, b_et3)
    for s in range(STEPS):
        aparts = _edge_sc(table, flat_w, dst_w, zeros_chunk)
        if s < STEPS - 1:
            h, table3 = _gru_table_tc(aparts, h, W_ih, W_hh, b_ih2, b_hh2,
                                      W_et, b_et3)
            table = table3.reshape(T * N, D)
        else:
            out_full = _gru_pool_tc(aparts, h, W_ih, W_hh, b_ih2, b_hh2,
                                    gid2, W1, b1_2, W2p, b2r)
    return out_full[:G, :1]


# speedup vs baseline: 1.4874x; 1.0106x over previous
import functools

import jax
import jax.numpy as jnp
from jax import lax
from jax.experimental import pallas as pl
from jax.experimental.pallas import tpu as pltpu
from jax.experimental.pallas import tpu_sc as plsc

N = 10000
E = 320000
D = 128
T = 4
G = 100
STEPS = 8
HID = 256

NC = 2
NS = 16
NW = NC * NS
EPW = E // NW
CH = 128
CPW = (EPW + CH - 1) // CH
PAD = CPW * CH - EPW
NACC = 10240
ZCH = NACC // NS // CH
ORT = NACC // NS

BN = 2000
NBLK = N // BN


def _table_tc(h, W_et, b_et):
    def body(h_ref, w_ref, b_ref, o_ref):
        o_ref[...] = lax.dot_general(
            h_ref[...], w_ref[0],
            (((1,), (1,)), ((), ())),
            preferred_element_type=jnp.float32) + b_ref[0]

    return pl.pallas_call(
        body,
        grid=(NBLK, T),
        in_specs=[
            pl.BlockSpec((BN, D), lambda i, t: (i, 0)),
            pl.BlockSpec((1, D, D), lambda i, t: (t, 0, 0)),
            pl.BlockSpec((1, 1, D), lambda i, t: (t, 0, 0)),
        ],
        out_specs=pl.BlockSpec((BN, D), lambda i, t: (t * NBLK + i, 0)),
        out_shape=jax.ShapeDtypeStruct((T * N, D), jnp.float32),
    )(h, W_et, b_et)


def _edge_sc(table, flat_w, dst_w, zeros_chunk):
    mesh = plsc.VectorSubcoreMesh(core_axis_name="c", subcore_axis_name="s")

    @functools.partial(
        pl.kernel,
        out_type=jax.ShapeDtypeStruct((NC, NACC, D), jnp.float32),
        mesh=mesh,
        scratch_types=[
            pltpu.VMEM((CPW, CH), jnp.int32),
            pltpu.VMEM((CPW, CH), jnp.int32),
            pltpu.VMEM((CH, D), jnp.float32),
            pltpu.VMEM_SHARED((NACC, D), jnp.float32),
            pltpu.SemaphoreType.DMA,
        ],
    )
    def run(table_h, flat_h, dst_h, zero_h, out_h, idx_v, dst_v, buf_v,
            acc_s, sem):
        c = lax.axis_index("c")
        s = lax.axis_index("s")
        wid = s * NC + c

        pltpu.sync_copy(zero_h, buf_v)
        for z in range(ZCH):
            pltpu.sync_copy(buf_v, acc_s.at[pl.ds(s * ORT + z * CH, CH)])
        plsc.subcore_barrier()

        pltpu.sync_copy(flat_h.at[wid], idx_v)
        pltpu.sync_copy(dst_h.at[wid], dst_v)

        def chunk(j, carry):
            pltpu.async_copy(table_h.at[idx_v.at[j]], buf_v, sem).wait()
            pltpu.sync_copy(buf_v, acc_s.at[dst_v.at[j]], add=True)
            return carry

        lax.fori_loop(0, CPW, chunk, 0)
        plsc.subcore_barrier()

        pltpu.sync_copy(acc_s.at[pl.ds(s * ORT, ORT)],
                        out_h.at[c, pl.ds(s * ORT, ORT)])

    return run(table, flat_w, dst_w, zeros_chunk)


def _gru_tc(aparts, h, W_ih, W_hh, b_ih2, b_hh2):
    def body(a_ref, h_ref, wi_ref, wh_ref, bi_ref, bh_ref, o_ref):
        a = a_ref[0] + a_ref[1]
        hb = h_ref[...]
        gi = lax.dot_general(a, wi_ref[...], (((1,), (1,)), ((), ())),
                             preferred_element_type=jnp.float32) + bi_ref[...]
        gh = lax.dot_general(hb, wh_ref[...], (((1,), (1,)), ((), ())),
                             preferred_element_type=jnp.float32) + bh_ref[...]
        r = jax.nn.sigmoid(gi[:, :D] + gh[:, :D])
        z = jax.nn.sigmoid(gi[:, D:2 * D] + gh[:, D:2 * D])
        n = jnp.tanh(gi[:, 2 * D:] + r * gh[:, 2 * D:])
        o_ref[...] = (1.0 - z) * n + z * hb

    return pl.pallas_call(
        body,
        grid=(NBLK,),
        in_specs=[
            pl.BlockSpec((NC, BN, D), lambda i: (0, i, 0)),
            pl.BlockSpec((BN, D), lambda i: (i, 0)),
            pl.BlockSpec((3 * D, D), lambda i: (0, 0)),
            pl.BlockSpec((3 * D, D), lambda i: (0, 0)),
            pl.BlockSpec((1, 3 * D), lambda i: (0, 0)),
            pl.BlockSpec((1, 3 * D), lambda i: (0, 0)),
        ],
        out_specs=pl.BlockSpec((BN, D), lambda i: (i, 0)),
        out_shape=jax.ShapeDtypeStruct((N, D), jnp.float32),
    )(aparts, h, W_ih, W_hh, b_ih2, b_hh2)


def _pool_mlp_tc(h, gid2, W1, b1_2, W2p, b2r):
    def body(h_ref, g_ref, w1_ref, b1_ref, w2_ref, b2_ref, o_ref, sums, cnts):
        i = pl.program_id(0)
        ids = g_ref[...]
        M = (ids == lax.broadcasted_iota(jnp.int32, (BN, D), 1)
             ).astype(jnp.float32)
        hb = h_ref[...]
        ps = lax.dot_general(M, hb, (((0,), (0,)), ((), ())),
                             preferred_element_type=jnp.float32)
        pc = lax.dot_general(M, jnp.ones((BN, D), jnp.float32),
                             (((0,), (0,)), ((), ())),
                             preferred_element_type=jnp.float32)

        @pl.when(i == 0)
        def _():
            sums[...] = ps
            cnts[...] = pc

        @pl.when(i > 0)
        def _():
            sums[...] += ps
            cnts[...] += pc

        @pl.when(i == NBLK - 1)
        def _():
            mean = sums[...] / jnp.maximum(cnts[...], 1.0)
            hid = jax.nn.relu(
                lax.dot_general(mean, w1_ref[...], (((1,), (1,)), ((), ())),
                                preferred_element_type=jnp.float32)
                + b1_ref[...])
            logit = lax.dot_general(hid, w2_ref[...], (((1,), (1,)), ((), ())),
                                    preferred_element_type=jnp.float32)
            o_ref[...] = jax.nn.sigmoid(logit + b2_ref[...])

    return pl.pallas_call(
        body,
        grid=(NBLK,),
        in_specs=[
            pl.BlockSpec((BN, D), lambda i: (i, 0)),
            pl.BlockSpec((BN, 1), lambda i: (i, 0)),
            pl.BlockSpec((HID, D), lambda i: (0, 0)),
            pl.BlockSpec((1, HID), lambda i: (0, 0)),
            pl.BlockSpec((D, HID), lambda i: (0, 0)),
            pl.BlockSpec((1, D), lambda i: (0, 0)),
        ],
        out_specs=pl.BlockSpec((D, D), lambda i: (0, 0)),
        out_shape=jax.ShapeDtypeStruct((D, D), jnp.float32),
        scratch_shapes=[
            pltpu.VMEM((D, D), jnp.float32),
            pltpu.VMEM((D, D), jnp.float32),
        ],
    )(h, gid2, W1, b1_2, W2p, b2r)


def _gru_table_tc(aparts, h, W_ih, W_hh, b_ih2, b_hh2, W_et, b_et3):
    def body(a_ref, h_ref, wi_ref, wh_ref, bi_ref, bh_ref, we_ref, be_ref,
             oh_ref, ot_ref):
        a = a_ref[0] + a_ref[1]
        hb = h_ref[...]
        gi = lax.dot_general(a, wi_ref[...], (((1,), (1,)), ((), ())),
                             preferred_element_type=jnp.float32) + bi_ref[...]
        gh = lax.dot_general(hb, wh_ref[...], (((1,), (1,)), ((), ())),
                             preferred_element_type=jnp.float32) + bh_ref[...]
        r = jax.nn.sigmoid(gi[:, :D] + gh[:, :D])
        z = jax.nn.sigmoid(gi[:, D:2 * D] + gh[:, D:2 * D])
        n = jnp.tanh(gi[:, 2 * D:] + r * gh[:, 2 * D:])
        hp = (1.0 - z) * n + z * hb
        oh_ref[...] = hp
        for t in range(T):
            ot_ref[t] = lax.dot_general(
                hp, we_ref[t], (((1,), (1,)), ((), ())),
                preferred_element_type=jnp.float32) + be_ref[t]

    return pl.pallas_call(
        body,
        grid=(NBLK,),
        in_specs=[
            pl.BlockSpec((NC, BN, D), lambda i: (0, i, 0)),
            pl.BlockSpec((BN, D), lambda i: (i, 0)),
            pl.BlockSpec((3 * D, D), lambda i: (0, 0)),
            pl.BlockSpec((3 * D, D), lambda i: (0, 0)),
            pl.BlockSpec((1, 3 * D), lambda i: (0, 0)),
            pl.BlockSpec((1, 3 * D), lambda i: (0, 0)),
            pl.BlockSpec((T, D, D), lambda i: (0, 0, 0)),
            pl.BlockSpec((T, 1, D), lambda i: (0, 0, 0)),
        ],
        out_specs=[
            pl.BlockSpec((BN, D), lambda i: (i, 0)),
            pl.BlockSpec((T, BN, D), lambda i: (0, i, 0)),
        ],
        out_shape=[
            jax.ShapeDtypeStruct((N, D), jnp.float32),
            jax.ShapeDtypeStruct((T, N, D), jnp.float32),
        ],
    )(aparts, h, W_ih, W_hh, b_ih2, b_hh2, W_et, b_et3)


def _gru_pool_tc(aparts, h, W_ih, W_hh, b_ih2, b_hh2, gid2, W1, b1_2, W2p,
                 b2r):
    def body(a_ref, h_ref, wi_ref, wh_ref, bi_ref, bh_ref, g_ref, w1_ref,
             b1_ref, w2_ref, b2_ref, o_ref, sums, cnts):
        i = pl.program_id(0)
        a = a_ref[0] + a_ref[1]
        hb = h_ref[...]
        gi = lax.dot_general(a, wi_ref[...], (((1,), (1,)), ((), ())),
                             preferred_element_type=jnp.float32) + bi_ref[...]
        gh = lax.dot_general(hb, wh_ref[...], (((1,), (1,)), ((), ())),
                             preferred_element_type=jnp.float32) + bh_ref[...]
        r = jax.nn.sigmoid(gi[:, :D] + gh[:, :D])
        z = jax.nn.sigmoid(gi[:, D:2 * D] + gh[:, D:2 * D])
        n = jnp.tanh(gi[:, 2 * D:] + r * gh[:, 2 * D:])
        hp = (1.0 - z) * n + z * hb

        M = (g_ref[...] == lax.broadcasted_iota(jnp.int32, (BN, D), 1)
             ).astype(jnp.float32)
        ps = lax.dot_general(M, hp, (((0,), (0,)), ((), ())),
                             preferred_element_type=jnp.float32)
        pc = lax.dot_general(M, jnp.ones((BN, D), jnp.float32),
                             (((0,), (0,)), ((), ())),
                             preferred_element_type=jnp.float32)

        @pl.when(i == 0)
        def _():
            sums[...] = ps
            cnts[...] = pc

        @pl.when(i > 0)
        def _():
            sums[...] += ps
            cnts[...] += pc

        @pl.when(i == NBLK - 1)
        def _():
            mean = sums[...] / jnp.maximum(cnts[...], 1.0)
            hid = jax.nn.relu(
                lax.dot_general(mean, w1_ref[...], (((1,), (1,)), ((), ())),
                                preferred_element_type=jnp.float32)
                + b1_ref[...])
            logit = lax.dot_general(hid, w2_ref[...], (((1,), (1,)), ((), ())),
                                    preferred_element_type=jnp.float32)
            o_ref[...] = jax.nn.sigmoid(logit + b2_ref[...])

    return pl.pallas_call(
        body,
        grid=(NBLK,),
        in_specs=[
            pl.BlockSpec((NC, BN, D), lambda i: (0, i, 0)),
            pl.BlockSpec((BN, D), lambda i: (i, 0)),
            pl.BlockSpec((3 * D, D), lambda i: (0, 0)),
            pl.BlockSpec((3 * D, D), lambda i: (0, 0)),
            pl.BlockSpec((1, 3 * D), lambda i: (0, 0)),
            pl.BlockSpec((1, 3 * D), lambda i: (0, 0)),
            pl.BlockSpec((BN, 1), lambda i: (i, 0)),
            pl.BlockSpec((HID, D), lambda i: (0, 0)),
            pl.BlockSpec((1, HID), lambda i: (0, 0)),
            pl.BlockSpec((D, HID), lambda i: (0, 0)),
            pl.BlockSpec((1, D), lambda i: (0, 0)),
        ],
        out_specs=pl.BlockSpec((D, D), lambda i: (0, 0)),
        out_shape=jax.ShapeDtypeStruct((D, D), jnp.float32),
        scratch_shapes=[
            pltpu.VMEM((D, D), jnp.float32),
            pltpu.VMEM((D, D), jnp.float32),
        ],
    )(aparts, h, W_ih, W_hh, b_ih2, b_hh2, gid2, W1, b1_2, W2p, b2r)


def kernel(x, edge_index, edge_types, graph_ids, W_et, b_et, W_ih, W_hh,
           b_ih, b_hh, W1, b1, W2, b2):
    src = edge_index[0]
    dst = edge_index[1]
    flat = edge_types * N + src

    flat_w = jnp.pad(flat.reshape(NW, EPW), ((0, 0), (0, PAD)),
                     constant_values=0).reshape(NW, CPW, CH)
    dst_w = jnp.pad(dst.reshape(NW, EPW), ((0, 0), (0, PAD)),
                    constant_values=N).reshape(NW, CPW, CH)
    zeros_chunk = jnp.zeros((CH, D), jnp.float32)

    b_et3 = b_et.reshape(T, 1, D)
    b_ih2 = b_ih.reshape(1, 3 * D)
    b_hh2 = b_hh.reshape(1, 3 * D)
    b1_2 = b1.reshape(1, HID)
    gid2 = graph_ids.reshape(N, 1)
    W2p = jnp.zeros((D, HID), jnp.float32).at[0].set(W2[0])
    b2r = jnp.broadcast_to(b2.reshape(1, 1), (1, D))

    h = x
    table = _table_tc(h, W_et, b_et3)
    for s in range(STEPS):
        aparts = _edge_sc(table, flat_w, dst_w, zeros_chunk)
        if s < STEPS - 1:
            h, table3 = _gru_table_tc(aparts, h, W_ih, W_hh, b_ih2, b_hh2,
                                      W_et, b_et3)
            table = table3.reshape(T * N, D)
        else:
            out_full = _gru_pool_tc(aparts, h, W_ih, W_hh, b_ih2, b_hh2,
                                    gid2, W1, b1_2, W2p, b2r)
    return out_full[:G, :1]
